# baseline (speedup 1.0000x reference)
import functools

import jax
import jax.numpy as jnp
from jax import lax
from jax.experimental import pallas as pl
from jax.experimental.pallas import tpu as pltpu
from jax.experimental.pallas import tpu_sc as plsc

F32 = jnp.float32
B, N, H, M = 8, 64, 256, 128
NN = B * N
NC, NS = 2, 16
NW = NC * NS


def _silu(x):
    return x * jax.nn.sigmoid(x)


def _prep_body(hf, cf, srcf, dstf, srcrow, dstrow, srcr, dstr, wm1,
               tab, wv, nnb, idxcat, odt, diffa, n2a):
    tab[:NN, :] = jnp.dot(hf[...], wm1[1:1 + H, :],
                          preferred_element_type=F32)
    tab[NN:, :] = jnp.dot(hf[...], wm1[1 + H:, :],
                          preferred_element_type=F32)
    e = srcf.shape[0]
    iota = lax.broadcasted_iota(jnp.int32, (e, N), 1)
    iota_t = lax.broadcasted_iota(jnp.int32, (N, e), 0)
    os_ = (srcf[...] == iota).astype(F32)
    od_ = (dstf[...] == iota).astype(F32)
    ost_ = (srcrow[...] == iota_t).astype(F32)
    odt_ = (dstrow[...] == iota_t).astype(F32)
    odt[...] = odt_
    cnt = jnp.dot(ost_, od_, preferred_element_type=F32)
    mult = jnp.sum(jnp.dot(os_, cnt, preferred_element_type=F32) * od_,
                   axis=1, keepdims=True)
    wv[...] = 1.0 / mult
    nnb[...] = jnp.sum(cnt, axis=1, keepdims=True)
    for b in range(B):
        cfb = cf[b * N:(b + 1) * N, :]
        db = (jnp.dot(os_, cfb, preferred_element_type=F32)
              - jnp.dot(od_, cfb, preferred_element_type=F32))
        diffa[b * e:(b + 1) * e, :] = db
        n2a[b * e:(b + 1) * e, :] = jnp.sqrt(
            jnp.sum(db * db, axis=1, keepdims=True))
    rows_s = jnp.concatenate([srcr[...]] * B, axis=0)
    rows_d = jnp.concatenate([dstr[...]] * B, axis=0)
    er = e // 128
    boff = (lax.broadcasted_iota(jnp.int32, (B * er, 128), 0) // er) * N
    idxcat[:B * er, :] = rows_s + boff
    idxcat[B * er:, :] = rows_d + boff + NN


def _edge_body(gs, gd, diffa, n2a, odt, cf, hf, wv, nnb, wn, bm1, wm2, bm2,
               wc1, bc1, wc2r, wh1h, wh1s, bh1, wh2, bh2, co, ho):
    cfv = cf[...]
    diff = diffa[...]
    n2 = n2a[...]
    g = gs[...] + gd[...]
    m = _silu(g + n2 * wn[...] + bm1[...])
    f = _silu(jnp.dot(m, wm2[...], preferred_element_type=F32) + bm2[...])
    cq = _silu(jnp.dot(f, wc1[...], preferred_element_type=F32) + bc1[...])
    c = jnp.sum(cq * wc2r[...], axis=1, keepdims=True)
    w = wv[...]
    odtv = odt[...]
    sum_h = jnp.dot(odtv, f * w, preferred_element_type=F32)
    sum_t = jnp.dot(odtv, diff * (c * w), preferred_element_type=F32)
    co[...] = cfv + sum_t / nnb[...]
    pre = _silu(jnp.dot(hf[...], wh1h[...], preferred_element_type=F32)
                + jnp.dot(sum_h, wh1s[...], preferred_element_type=F32)
                + bh1[...])
    ho[...] = jnp.dot(pre, wh2[...], preferred_element_type=F32) + bh2[...]


def _mesh():
    return plsc.VectorSubcoreMesh(core_axis_name="c", subcore_axis_name="s",
                                  num_cores=NC, num_subcores=NS)


def _sc_gather(tab, idxcat, ef):
    chunk = ef // NW
    nj = chunk // 128
    nr = ef // 128

    @functools.partial(
        pl.kernel, mesh=_mesh(),
        out_type=(jax.ShapeDtypeStruct((ef, M), F32),
                  jax.ShapeDtypeStruct((ef, M), F32)),
        scratch_types=[pltpu.VMEM((2 * nj, 128), jnp.int32),
                       pltpu.VMEM((chunk, M), F32),
                       pltpu.VMEM((chunk, M), F32),
                       pltpu.SemaphoreType.DMA,
                       pltpu.SemaphoreType.DMA,
                       pltpu.SemaphoreType.DMA,
                       pltpu.SemaphoreType.DMA,
                       pltpu.SemaphoreType.DMA],
    )
    def k(tab_hbm, idx_hbm, gs_hbm, gd_hbm,
          idx, bufs, bufd, sem0, sem1, sem2, sem3, wsem):
        cid = lax.axis_index("c")
        sid = lax.axis_index("s")
        wid = sid * NC + cid
        base = wid * chunk
        pltpu.sync_copy(idx_hbm.at[pl.ds(nj * wid, nj)],
                        idx.at[pl.ds(0, nj)])
        pltpu.sync_copy(idx_hbm.at[pl.ds(nr + nj * wid, nj)],
                        idx.at[pl.ds(nj, nj)])
        gsems = [sem0, sem1, sem2, sem3]
        gets = [pltpu.async_copy(tab_hbm.at[idx.at[j]],
                                 (bufs if j < nj else bufd)
                                 .at[pl.ds((j % nj) * 128, 128)],
                                 gsems[j])
                for j in range(2 * nj)]
        puts = []
        for j in range(2 * nj):
            gets[j].wait()
            dst = gs_hbm if j < nj else gd_hbm
            buf = bufs if j < nj else bufd
            puts.append(pltpu.async_copy(
                buf.at[pl.ds((j % nj) * 128, 128)],
                dst.at[pl.ds(base + (j % nj) * 128, 128)], wsem))
        for p in puts:
            p.wait()

    return k(tab, idxcat)


def kernel(coords, hidden, edges, Wm1, bm1, Wm2, bm2, Wc1, bc1, Wc2,
           Wh1, bh1, Wh2, bh2):
    e = edges.shape[1]
    ef = B * e
    cf = coords.reshape(NN, 3).astype(F32)
    hf = hidden.reshape(NN, H).astype(F32)

    srcf = edges[0].astype(jnp.int32)[:, None]
    dstf = edges[1].astype(jnp.int32)[:, None]
    srcrow = edges[0].astype(jnp.int32)[None, :]
    dstrow = edges[1].astype(jnp.int32)[None, :]
    srcr = edges[0].astype(jnp.int32).reshape(e // 128, 128)
    dstr = edges[1].astype(jnp.int32).reshape(e // 128, 128)

    tab, wv, nnb, idxcat, odt, diffa, n2a = pl.pallas_call(
        _prep_body,
        out_shape=(jax.ShapeDtypeStruct((2 * NN, M), F32),
                   jax.ShapeDtypeStruct((e, 1), F32),
                   jax.ShapeDtypeStruct((N, 1), F32),
                   jax.ShapeDtypeStruct((2 * ef // 128, 128), jnp.int32),
                   jax.ShapeDtypeStruct((N, e), F32),
                   jax.ShapeDtypeStruct((ef, 3), F32),
                   jax.ShapeDtypeStruct((ef, 1), F32)),
    )(hf, cf, srcf, dstf, srcrow, dstrow, srcr, dstr, Wm1)

    gs, gd = _sc_gather(tab, idxcat, ef)

    full = lambda shape: pl.BlockSpec(shape, lambda i: tuple(0 for _ in shape))
    co, ho = pl.pallas_call(
        _edge_body,
        grid=(B,),
        in_specs=[pl.BlockSpec((e, M), lambda i: (i, 0)),
                  pl.BlockSpec((e, M), lambda i: (i, 0)),
                  pl.BlockSpec((e, 3), lambda i: (i, 0)),
                  pl.BlockSpec((e, 1), lambda i: (i, 0)),
                  full((N, e)),
                  pl.BlockSpec((N, 3), lambda i: (i, 0)),
                  pl.BlockSpec((N, H), lambda i: (i, 0)),
                  full((e, 1)), full((N, 1)),
                  full((1, M)), full((1, M)), full((M, M)), full((1, M)),
                  full((M, M)), full((1, M)), full((1, M)),
                  full((H, M)), full((M, M)), full((1, M)),
                  full((M, H)), full((1, H))],
        out_specs=(pl.BlockSpec((N, 3), lambda i: (i, 0)),
                   pl.BlockSpec((N, H), lambda i: (i, 0))),
        out_shape=(jax.ShapeDtypeStruct((NN, 3), F32),
                   jax.ShapeDtypeStruct((NN, H), F32)),
    )(gs, gd, diffa, n2a, odt, cf, hf, wv, nnb, Wm1[0:1], bm1[None, :], Wm2,
      bm2[None, :], Wc1, bc1[None, :], Wc2.reshape(1, M),
      Wh1[:H], Wh1[H:], bh1[None, :], Wh2, bh2[None, :])

    coords_out = co.reshape(B, N, 3)
    hidden_out = ho.reshape(B, N, H)
    return coords_out, hidden_out


# speedup vs baseline: 1.1386x; 1.1386x over previous
import functools

import jax
import jax.numpy as jnp
from jax import lax
from jax.experimental import pallas as pl
from jax.experimental.pallas import tpu as pltpu
from jax.experimental.pallas import tpu_sc as plsc

F32 = jnp.float32
B, N, H, M = 8, 64, 256, 128
NN = B * N
NC, NS = 2, 16
NW = NC * NS


def _silu(x):
    return x * jax.nn.sigmoid(x)


def _tab_body(h3, srcr, dstr, wm1, tab, idxcat):
    ws = wm1[1:1 + H, :]
    wd = wm1[1 + H:, :]
    for b in range(B):
        hb = h3[b]
        tab[b * N:(b + 1) * N, :] = jnp.dot(hb, ws,
                                            preferred_element_type=F32)
        tab[NN + b * N:NN + (b + 1) * N, :] = jnp.dot(
            hb, wd, preferred_element_type=F32)
    e = srcr.shape[0] * 128
    er = e // 128
    rows_s = jnp.concatenate([srcr[...]] * B, axis=0)
    rows_d = jnp.concatenate([dstr[...]] * B, axis=0)
    boff = (lax.broadcasted_iota(jnp.int32, (B * er, 128), 0) // er) * N
    idxcat[:B * er, :] = rows_s + boff
    idxcat[B * er:, :] = rows_d + boff + NN


def _cnt_body(srcf, dstf, srcrow, dstrow, wv, nnb, odt):
    e = srcf.shape[0]
    iota = lax.broadcasted_iota(jnp.int32, (e, N), 1)
    iota_t = lax.broadcasted_iota(jnp.int32, (N, e), 0)
    os_ = (srcf[...] == iota).astype(F32)
    od_ = (dstf[...] == iota).astype(F32)
    ost_ = (srcrow[...] == iota_t).astype(F32)
    odt_ = (dstrow[...] == iota_t).astype(F32)
    odt[...] = odt_
    cnt = jnp.dot(ost_, od_, preferred_element_type=F32)
    mult = jnp.sum(jnp.dot(os_, cnt, preferred_element_type=F32) * od_,
                   axis=1, keepdims=True)
    wv[...] = 1.0 / mult
    nnb[...] = jnp.sum(cnt, axis=1, keepdims=True)


def _edge_body(gs, gd, srcf, dstf, odt, c3, h3, wv, nnb, wn, bm1, wm2, bm2,
               wc1, bc1, wc2r, wh1h, wh1s, bh1, wh2, bh2, co, ho):
    e = srcf.shape[0]
    iota = lax.broadcasted_iota(jnp.int32, (e, N), 1)
    os_ = (srcf[...] == iota).astype(F32)
    od_ = (dstf[...] == iota).astype(F32)
    cb = c3[0]
    diff = (jnp.dot(os_, cb, preferred_element_type=F32)
            - jnp.dot(od_, cb, preferred_element_type=F32))
    d2 = jnp.sum(diff * diff, axis=1, keepdims=True)
    n2 = jnp.sqrt(d2)
    g = gs[...] + gd[...]
    m = _silu(g + n2 * wn[...] + bm1[...])
    f = _silu(jnp.dot(m, wm2[...], preferred_element_type=F32) + bm2[...])
    cq = _silu(jnp.dot(f, wc1[...], preferred_element_type=F32) + bc1[...])
    c = jnp.sum(cq * wc2r[...], axis=1, keepdims=True)
    w = wv[...]
    odtv = odt[...]
    sum_h = jnp.dot(odtv, f * w, preferred_element_type=F32)
    sum_t = jnp.dot(odtv, diff * (c * w), preferred_element_type=F32)
    co[0] = cb + sum_t / nnb[...]
    pre = _silu(jnp.dot(h3[0], wh1h[...], preferred_element_type=F32)
                + jnp.dot(sum_h, wh1s[...], preferred_element_type=F32)
                + bh1[...])
    ho[0] = jnp.dot(pre, wh2[...], preferred_element_type=F32) + bh2[...]


def _mesh():
    return plsc.VectorSubcoreMesh(core_axis_name="c", subcore_axis_name="s",
                                  num_cores=NC, num_subcores=NS)


def _sc_gather(tab, idxcat, ef):
    chunk = ef // NW
    nj = chunk // 128
    nr = ef // 128

    @functools.partial(
        pl.kernel, mesh=_mesh(),
        out_type=(jax.ShapeDtypeStruct((ef, M), F32),
                  jax.ShapeDtypeStruct((ef, M), F32)),
        scratch_types=[pltpu.VMEM((2 * nj, 128), jnp.int32),
                       pltpu.VMEM((chunk, M), F32),
                       pltpu.VMEM((chunk, M), F32),
                       pltpu.SemaphoreType.DMA,
                       pltpu.SemaphoreType.DMA,
                       pltpu.SemaphoreType.DMA,
                       pltpu.SemaphoreType.DMA,
                       pltpu.SemaphoreType.DMA],
    )
    def k(tab_hbm, idx_hbm, gs_hbm, gd_hbm,
          idx, bufs, bufd, sem0, sem1, sem2, sem3, wsem):
        cid = lax.axis_index("c")
        sid = lax.axis_index("s")
        wid = sid * NC + cid
        base = wid * chunk
        pltpu.sync_copy(idx_hbm.at[pl.ds(nj * wid, nj)],
                        idx.at[pl.ds(0, nj)])
        pltpu.sync_copy(idx_hbm.at[pl.ds(nr + nj * wid, nj)],
                        idx.at[pl.ds(nj, nj)])
        gsems = [sem0, sem1, sem2, sem3]
        gets = [pltpu.async_copy(tab_hbm.at[idx.at[j]],
                                 (bufs if j < nj else bufd)
                                 .at[pl.ds((j % nj) * 128, 128)],
                                 gsems[j])
                for j in range(2 * nj)]
        puts = []
        for j in range(2 * nj):
            gets[j].wait()
            dst = gs_hbm if j < nj else gd_hbm
            buf = bufs if j < nj else bufd
            puts.append(pltpu.async_copy(
                buf.at[pl.ds((j % nj) * 128, 128)],
                dst.at[pl.ds(base + (j % nj) * 128, 128)], wsem))
        for p in puts:
            p.wait()

    return k(tab, idxcat)


def kernel(coords, hidden, edges, Wm1, bm1, Wm2, bm2, Wc1, bc1, Wc2,
           Wh1, bh1, Wh2, bh2):
    e = edges.shape[1]
    ef = B * e

    srcf = edges[0].astype(jnp.int32)[:, None]
    dstf = edges[1].astype(jnp.int32)[:, None]
    srcrow = edges[0].astype(jnp.int32)[None, :]
    dstrow = edges[1].astype(jnp.int32)[None, :]
    srcr = edges[0].astype(jnp.int32).reshape(e // 128, 128)
    dstr = edges[1].astype(jnp.int32).reshape(e // 128, 128)

    tab, idxcat = pl.pallas_call(
        _tab_body,
        out_shape=(jax.ShapeDtypeStruct((2 * NN, M), F32),
                   jax.ShapeDtypeStruct((2 * ef // 128, 128), jnp.int32)),
    )(hidden, srcr, dstr, Wm1)

    gs, gd = _sc_gather(tab, idxcat, ef)

    wv, nnb, odt = pl.pallas_call(
        _cnt_body,
        out_shape=(jax.ShapeDtypeStruct((e, 1), F32),
                   jax.ShapeDtypeStruct((N, 1), F32),
                   jax.ShapeDtypeStruct((N, e), F32)),
    )(srcf, dstf, srcrow, dstrow)

    full = lambda shape: pl.BlockSpec(shape, lambda i: tuple(0 for _ in shape))
    co, ho = pl.pallas_call(
        _edge_body,
        grid=(B,),
        in_specs=[pl.BlockSpec((e, M), lambda i: (i, 0)),
                  pl.BlockSpec((e, M), lambda i: (i, 0)),
                  full((e, 1)), full((e, 1)),
                  full((N, e)),
                  pl.BlockSpec((1, N, 3), lambda i: (i, 0, 0)),
                  pl.BlockSpec((1, N, H), lambda i: (i, 0, 0)),
                  full((e, 1)), full((N, 1)),
                  full((1, M)), full((1, M)), full((M, M)), full((1, M)),
                  full((M, M)), full((1, M)), full((1, M)),
                  full((H, M)), full((M, M)), full((1, M)),
                  full((M, H)), full((1, H))],
        out_specs=(pl.BlockSpec((1, N, 3), lambda i: (i, 0, 0)),
                   pl.BlockSpec((1, N, H), lambda i: (i, 0, 0))),
        out_shape=(jax.ShapeDtypeStruct((B, N, 3), F32),
                   jax.ShapeDtypeStruct((B, N, H), F32)),
    )(gs, gd, srcf, dstf, odt, coords, hidden, wv, nnb,
      Wm1[0:1], bm1[None, :], Wm2, bm2[None, :], Wc1, bc1[None, :],
      Wc2.reshape(1, M), Wh1[:H], Wh1[H:], bh1[None, :], Wh2, bh2[None, :])

    return co, ho


# speedup vs baseline: 1.1760x; 1.0328x over previous
import functools

import jax
import jax.numpy as jnp
from jax import lax
from jax.experimental import pallas as pl
from jax.experimental.pallas import tpu as pltpu
from jax.experimental.pallas import tpu_sc as plsc

F32 = jnp.float32
B, N, H, M = 8, 64, 256, 128
NN = B * N
NC, NS = 2, 16
NW = NC * NS


def _silu(x):
    return x * jax.nn.sigmoid(x)


def _tab_body(h3, srcr, dstr, wm1, tab, idxcat):
    ws = wm1[1:1 + H, :]
    wd = wm1[1 + H:, :]
    for b in range(B):
        hb = h3[b]
        tab[b * N:(b + 1) * N, :] = jnp.dot(hb, ws,
                                            preferred_element_type=F32)
        tab[NN + b * N:NN + (b + 1) * N, :] = jnp.dot(
            hb, wd, preferred_element_type=F32)
    e = srcr.shape[0] * 128
    er = e // 128
    rows_s = jnp.concatenate([srcr[...]] * B, axis=0)
    rows_d = jnp.concatenate([dstr[...]] * B, axis=0)
    boff = (lax.broadcasted_iota(jnp.int32, (B * er, 128), 0) // er) * N
    idxcat[:B * er, :] = rows_s + boff
    idxcat[B * er:, :] = rows_d + boff + NN


def _cnt_body(srcf, dstf, srcrow, dstrow, wv, nnb, odt):
    e = srcf.shape[0]
    iota = lax.broadcasted_iota(jnp.int32, (e, N), 1)
    iota_t = lax.broadcasted_iota(jnp.int32, (N, e), 0)
    os_ = (srcf[...] == iota).astype(F32)
    od_ = (dstf[...] == iota).astype(F32)
    ost_ = (srcrow[...] == iota_t).astype(F32)
    odt_ = (dstrow[...] == iota_t).astype(F32)
    odt[...] = odt_
    cnt = jnp.dot(ost_, od_, preferred_element_type=F32)
    mult = jnp.sum(jnp.dot(os_, cnt, preferred_element_type=F32) * od_,
                   axis=1, keepdims=True)
    wv[...] = 1.0 / mult
    nnb[...] = jnp.sum(cnt, axis=1, keepdims=True)


def _edge_body(gref, srcf, dstf, odt, c3, h3, wv, nnb, wn, bm1, wm2, bm2,
               wc1, bc1, wc2r, wh1h, wh1s, bh1, wh2, bh2, co, ho):
    e = srcf.shape[0]
    iota = lax.broadcasted_iota(jnp.int32, (e, N), 1)
    os_ = (srcf[...] == iota).astype(F32)
    od_ = (dstf[...] == iota).astype(F32)
    cb = c3[0]
    diff = (jnp.dot(os_, cb, preferred_element_type=F32)
            - jnp.dot(od_, cb, preferred_element_type=F32))
    d2 = jnp.sum(diff * diff, axis=1, keepdims=True)
    n2 = jnp.sqrt(d2)
    g = gref[...]
    m = _silu(g + n2 * wn[...] + bm1[...])
    f = _silu(jnp.dot(m, wm2[...], preferred_element_type=F32) + bm2[...])
    cq = _silu(jnp.dot(f, wc1[...], preferred_element_type=F32) + bc1[...])
    c = jnp.sum(cq * wc2r[...], axis=1, keepdims=True)
    w = wv[...]
    odtv = odt[...]
    sum_h = jnp.dot(odtv, f * w, preferred_element_type=F32)
    sum_t = jnp.dot(odtv, diff * (c * w), preferred_element_type=F32)
    co[0] = cb + sum_t / nnb[...]
    pre = _silu(jnp.dot(h3[0], wh1h[...], preferred_element_type=F32)
                + jnp.dot(sum_h, wh1s[...], preferred_element_type=F32)
                + bh1[...])
    ho[0] = jnp.dot(pre, wh2[...], preferred_element_type=F32) + bh2[...]


def _mesh():
    return plsc.VectorSubcoreMesh(core_axis_name="c", subcore_axis_name="s",
                                  num_cores=NC, num_subcores=NS)


def _sc_gather(tab, idxcat, ef):
    chunk = ef // NW
    nj = chunk // 128
    nr = ef // 128

    @functools.partial(
        pl.kernel, mesh=_mesh(),
        out_type=jax.ShapeDtypeStruct((ef, M), F32),
        scratch_types=[pltpu.VMEM((2 * nj, 128), jnp.int32),
                       pltpu.VMEM((chunk, M), F32),
                       pltpu.VMEM((chunk, M), F32),
                       pltpu.SemaphoreType.DMA,
                       pltpu.SemaphoreType.DMA,
                       pltpu.SemaphoreType.DMA,
                       pltpu.SemaphoreType.DMA,
                       pltpu.SemaphoreType.DMA],
    )
    def k(tab_hbm, idx_hbm, g_hbm,
          idx, bufs, bufd, sem0, sem1, sem2, sem3, wsem):
        cid = lax.axis_index("c")
        sid = lax.axis_index("s")
        wid = sid * NC + cid
        base = wid * chunk
        pltpu.sync_copy(idx_hbm.at[pl.ds(nj * wid, nj)],
                        idx.at[pl.ds(0, nj)])
        pltpu.sync_copy(idx_hbm.at[pl.ds(nr + nj * wid, nj)],
                        idx.at[pl.ds(nj, nj)])
        gsems = [sem0, sem1, sem2, sem3]
        gets = [pltpu.async_copy(tab_hbm.at[idx.at[j]],
                                 (bufs if j < nj else bufd)
                                 .at[pl.ds((j % nj) * 128, 128)],
                                 gsems[j])
                for j in range(2 * nj)]
        puts = []
        for j in range(nj):
            gets[j].wait()
            gets[nj + j].wait()

            def body(r, _, j=j):
                row = j * 128 + r
                for l in range(M // 16):
                    sl = pl.ds(l * 16, 16)
                    bufs[row, sl] = bufs[row, sl] + bufd[row, sl]
                return 0

            lax.fori_loop(0, 128, body, 0)
            puts.append(pltpu.async_copy(
                bufs.at[pl.ds(j * 128, 128)],
                g_hbm.at[pl.ds(base + j * 128, 128)], wsem))
        for p in puts:
            p.wait()

    return k(tab, idxcat)


def kernel(coords, hidden, edges, Wm1, bm1, Wm2, bm2, Wc1, bc1, Wc2,
           Wh1, bh1, Wh2, bh2):
    e = edges.shape[1]
    ef = B * e

    srcf = edges[0].astype(jnp.int32)[:, None]
    dstf = edges[1].astype(jnp.int32)[:, None]
    srcrow = edges[0].astype(jnp.int32)[None, :]
    dstrow = edges[1].astype(jnp.int32)[None, :]
    srcr = edges[0].astype(jnp.int32).reshape(e // 128, 128)
    dstr = edges[1].astype(jnp.int32).reshape(e // 128, 128)

    tab, idxcat = pl.pallas_call(
        _tab_body,
        out_shape=(jax.ShapeDtypeStruct((2 * NN, M), F32),
                   jax.ShapeDtypeStruct((2 * ef // 128, 128), jnp.int32)),
    )(hidden, srcr, dstr, Wm1)

    g = _sc_gather(tab, idxcat, ef)

    wv, nnb, odt = pl.pallas_call(
        _cnt_body,
        out_shape=(jax.ShapeDtypeStruct((e, 1), F32),
                   jax.ShapeDtypeStruct((N, 1), F32),
                   jax.ShapeDtypeStruct((N, e), F32)),
    )(srcf, dstf, srcrow, dstrow)

    full = lambda shape: pl.BlockSpec(shape, lambda i: tuple(0 for _ in shape))
    co, ho = pl.pallas_call(
        _edge_body,
        grid=(B,),
        in_specs=[pl.BlockSpec((e, M), lambda i: (i, 0)),
                  full((e, 1)), full((e, 1)),
                  full((N, e)),
                  pl.BlockSpec((1, N, 3), lambda i: (i, 0, 0)),
                  pl.BlockSpec((1, N, H), lambda i: (i, 0, 0)),
                  full((e, 1)), full((N, 1)),
                  full((1, M)), full((1, M)), full((M, M)), full((1, M)),
                  full((M, M)), full((1, M)), full((1, M)),
                  full((H, M)), full((M, M)), full((1, M)),
                  full((M, H)), full((1, H))],
        out_specs=(pl.BlockSpec((1, N, 3), lambda i: (i, 0, 0)),
                   pl.BlockSpec((1, N, H), lambda i: (i, 0, 0))),
        out_shape=(jax.ShapeDtypeStruct((B, N, 3), F32),
                   jax.ShapeDtypeStruct((B, N, H), F32)),
    )(g, srcf, dstf, odt, coords, hidden, wv, nnb,
      Wm1[0:1], bm1[None, :], Wm2, bm2[None, :], Wc1, bc1[None, :],
      Wc2.reshape(1, M), Wh1[:H], Wh1[H:], bh1[None, :], Wh2, bh2[None, :])

    return co, ho


# speedup vs baseline: 1.2453x; 1.0589x over previous
import functools

import jax
import jax.numpy as jnp
from jax import lax
from jax.experimental import pallas as pl
from jax.experimental.pallas import tpu as pltpu
from jax.experimental.pallas import tpu_sc as plsc

F32 = jnp.float32
B, N, H, M = 8, 64, 256, 128
NN = B * N
NC, NS = 2, 16
NW = NC * NS


def _silu(x):
    return x * jax.nn.sigmoid(x)


def _tab_body(h3, srcr, dstr, wm1, tab, idxcat):
    ws = wm1[1:1 + H, :]
    wd = wm1[1 + H:, :]
    for b in range(B):
        hb = h3[b]
        tab[b * N:(b + 1) * N, :] = jnp.dot(hb, ws,
                                            preferred_element_type=F32)
        tab[NN + b * N:NN + (b + 1) * N, :] = jnp.dot(
            hb, wd, preferred_element_type=F32)
    e = srcr.shape[0] * 128
    er = e // 128
    rows_s = jnp.concatenate([srcr[...]] * B, axis=0)
    rows_d = jnp.concatenate([dstr[...]] * B, axis=0)
    boff = (lax.broadcasted_iota(jnp.int32, (B * er, 128), 0) // er) * N
    idxcat[:B * er, :] = rows_s + boff
    idxcat[B * er:, :] = rows_d + boff + NN


def _edge_body(gref, srcf, dstf, srcrow, dstrow, c3, h3, wn, bm1, wm2, bm2,
               wc1, bc1, wc2r, wh1h, wh1s, bh1, wh2, bh2, co, ho):
    e = srcf.shape[0]
    iota = lax.broadcasted_iota(jnp.int32, (e, N), 1)
    iota_t = lax.broadcasted_iota(jnp.int32, (N, e), 0)
    os_ = (srcf[...] == iota).astype(F32)
    od_ = (dstf[...] == iota).astype(F32)
    ost_ = (srcrow[...] == iota_t).astype(F32)
    odt_ = (dstrow[...] == iota_t).astype(F32)
    cnt = jnp.dot(ost_, od_, preferred_element_type=F32)
    mult = jnp.sum(jnp.dot(os_, cnt, preferred_element_type=F32) * od_,
                   axis=1, keepdims=True)
    w = 1.0 / mult
    nnb = jnp.sum(cnt, axis=1, keepdims=True)
    for b in range(B):
        gb = gref[b * e:(b + 1) * e, :]
        cb = c3[b]
        diff = (jnp.dot(os_, cb, preferred_element_type=F32)
                - jnp.dot(od_, cb, preferred_element_type=F32))
        d2 = jnp.sum(diff * diff, axis=1, keepdims=True)
        n2 = jnp.sqrt(d2)
        m = _silu(gb + n2 * wn[...] + bm1[...])
        f = _silu(jnp.dot(m, wm2[...], preferred_element_type=F32)
                  + bm2[...])
        cq = _silu(jnp.dot(f, wc1[...], preferred_element_type=F32)
                   + bc1[...])
        c = jnp.sum(cq * wc2r[...], axis=1, keepdims=True)
        sum_h = jnp.dot(odt_, f * w, preferred_element_type=F32)
        sum_t = jnp.dot(odt_, diff * (c * w), preferred_element_type=F32)
        co[b] = cb + sum_t / nnb
        pre = _silu(jnp.dot(h3[b], wh1h[...], preferred_element_type=F32)
                    + jnp.dot(sum_h, wh1s[...], preferred_element_type=F32)
                    + bh1[...])
        ho[b] = jnp.dot(pre, wh2[...], preferred_element_type=F32) + bh2[...]


def _mesh():
    return plsc.VectorSubcoreMesh(core_axis_name="c", subcore_axis_name="s",
                                  num_cores=NC, num_subcores=NS)


def _sc_gather(tab, idxcat, ef):
    chunk = ef // NW
    nj = chunk // 128
    nr = ef // 128

    @functools.partial(
        pl.kernel, mesh=_mesh(),
        out_type=jax.ShapeDtypeStruct((ef, M), F32),
        scratch_types=[pltpu.VMEM((2 * nj, 128), jnp.int32),
                       pltpu.VMEM((chunk, M), F32),
                       pltpu.VMEM((chunk, M), F32),
                       pltpu.SemaphoreType.DMA,
                       pltpu.SemaphoreType.DMA,
                       pltpu.SemaphoreType.DMA,
                       pltpu.SemaphoreType.DMA,
                       pltpu.SemaphoreType.DMA],
    )
    def k(tab_hbm, idx_hbm, g_hbm,
          idx, bufs, bufd, sem0, sem1, sem2, sem3, wsem):
        cid = lax.axis_index("c")
        sid = lax.axis_index("s")
        wid = sid * NC + cid
        base = wid * chunk
        pltpu.sync_copy(idx_hbm.at[pl.ds(nj * wid, nj)],
                        idx.at[pl.ds(0, nj)])
        pltpu.sync_copy(idx_hbm.at[pl.ds(nr + nj * wid, nj)],
                        idx.at[pl.ds(nj, nj)])
        gsems = [sem0, sem1, sem2, sem3]
        gets = [pltpu.async_copy(tab_hbm.at[idx.at[j]],
                                 (bufs if j < nj else bufd)
                                 .at[pl.ds((j % nj) * 128, 128)],
                                 gsems[j])
                for j in range(2 * nj)]
        puts = []
        for j in range(nj):
            gets[j].wait()
            gets[nj + j].wait()

            def body(r, _, j=j):
                row = j * 128 + r
                for l in range(M // 16):
                    sl = pl.ds(l * 16, 16)
                    bufs[row, sl] = bufs[row, sl] + bufd[row, sl]
                return 0

            lax.fori_loop(0, 128, body, 0)
            puts.append(pltpu.async_copy(
                bufs.at[pl.ds(j * 128, 128)],
                g_hbm.at[pl.ds(base + j * 128, 128)], wsem))
        for p in puts:
            p.wait()

    return k(tab, idxcat)


def kernel(coords, hidden, edges, Wm1, bm1, Wm2, bm2, Wc1, bc1, Wc2,
           Wh1, bh1, Wh2, bh2):
    e = edges.shape[1]
    ef = B * e

    srcf = edges[0].astype(jnp.int32)[:, None]
    dstf = edges[1].astype(jnp.int32)[:, None]
    srcrow = edges[0].astype(jnp.int32)[None, :]
    dstrow = edges[1].astype(jnp.int32)[None, :]
    srcr = edges[0].astype(jnp.int32).reshape(e // 128, 128)
    dstr = edges[1].astype(jnp.int32).reshape(e // 128, 128)

    tab, idxcat = pl.pallas_call(
        _tab_body,
        out_shape=(jax.ShapeDtypeStruct((2 * NN, M), F32),
                   jax.ShapeDtypeStruct((2 * ef // 128, 128), jnp.int32)),
    )(hidden, srcr, dstr, Wm1)

    g = _sc_gather(tab, idxcat, ef)

    co, ho = pl.pallas_call(
        _edge_body,
        out_shape=(jax.ShapeDtypeStruct((B, N, 3), F32),
                   jax.ShapeDtypeStruct((B, N, H), F32)),
    )(g, srcf, dstf, srcrow, dstrow, coords, hidden,
      Wm1[0:1], bm1[None, :], Wm2, bm2[None, :], Wc1, bc1[None, :],
      Wc2.reshape(1, M), Wh1[:H], Wh1[H:], bh1[None, :], Wh2, bh2[None, :])

    return co, ho


# speedup vs baseline: 1.2602x; 1.0119x over previous
import functools

import jax
import jax.numpy as jnp
from jax import lax
from jax.experimental import pallas as pl
from jax.experimental.pallas import tpu as pltpu
from jax.experimental.pallas import tpu_sc as plsc

F32 = jnp.float32
B, N, H, M = 8, 64, 256, 128
NN = B * N
NC, NS = 2, 16
NW = NC * NS


def _silu(x):
    return x * jax.nn.sigmoid(x)


def _tab_body(h3, wm1, tab):
    ws = wm1[1:1 + H, :]
    wd = wm1[1 + H:, :]
    for b in range(B):
        hb = h3[b]
        tab[b * N:(b + 1) * N, :] = jnp.dot(hb, ws,
                                            preferred_element_type=F32)
        tab[NN + b * N:NN + (b + 1) * N, :] = jnp.dot(
            hb, wd, preferred_element_type=F32)


def _edge_body(gref, srcf, dstf, srcrow, dstrow, c3, h3, wn, bm1, wm2, bm2,
               wc1, bc1, wc2r, wh1h, wh1s, bh1, wh2, bh2, co, ho):
    e = srcf.shape[0]
    iota = lax.broadcasted_iota(jnp.int32, (e, N), 1)
    iota_t = lax.broadcasted_iota(jnp.int32, (N, e), 0)
    os_ = (srcf[...] == iota).astype(F32)
    od_ = (dstf[...] == iota).astype(F32)
    ost_ = (srcrow[...] == iota_t).astype(F32)
    odt_ = (dstrow[...] == iota_t).astype(F32)
    cnt = jnp.dot(ost_, od_, preferred_element_type=F32)
    mult = jnp.sum(jnp.dot(os_, cnt, preferred_element_type=F32) * od_,
                   axis=1, keepdims=True)
    w = 1.0 / mult
    nnb = jnp.sum(cnt, axis=1, keepdims=True)
    for b in range(B):
        gb = gref[b * e:(b + 1) * e, :]
        cb = c3[b]
        diff = (jnp.dot(os_, cb, preferred_element_type=F32)
                - jnp.dot(od_, cb, preferred_element_type=F32))
        d2 = jnp.sum(diff * diff, axis=1, keepdims=True)
        n2 = jnp.sqrt(d2)
        m = _silu(gb + n2 * wn[...] + bm1[...])
        f = _silu(jnp.dot(m, wm2[...], preferred_element_type=F32)
                  + bm2[...])
        cq = _silu(jnp.dot(f, wc1[...], preferred_element_type=F32)
                   + bc1[...])
        c = jnp.sum(cq * wc2r[...], axis=1, keepdims=True)
        sum_h = jnp.dot(odt_, f * w, preferred_element_type=F32)
        sum_t = jnp.dot(odt_, diff * (c * w), preferred_element_type=F32)
        co[b] = cb + sum_t / nnb
        pre = _silu(jnp.dot(h3[b], wh1h[...], preferred_element_type=F32)
                    + jnp.dot(sum_h, wh1s[...], preferred_element_type=F32)
                    + bh1[...])
        ho[b] = jnp.dot(pre, wh2[...], preferred_element_type=F32) + bh2[...]


def _mesh():
    return plsc.VectorSubcoreMesh(core_axis_name="c", subcore_axis_name="s",
                                  num_cores=NC, num_subcores=NS)


def _sc_gather(tab, idxcat, ef):
    chunk = ef // NW
    nj = chunk // 128
    nr = ef // 128

    @functools.partial(
        pl.kernel, mesh=_mesh(),
        out_type=jax.ShapeDtypeStruct((ef, M), F32),
        scratch_types=[pltpu.VMEM((2 * nj, 128), jnp.int32),
                       pltpu.VMEM((chunk, M), F32),
                       pltpu.VMEM((chunk, M), F32),
                       pltpu.SemaphoreType.DMA,
                       pltpu.SemaphoreType.DMA,
                       pltpu.SemaphoreType.DMA,
                       pltpu.SemaphoreType.DMA,
                       pltpu.SemaphoreType.DMA],
    )
    def k(tab_hbm, idx_hbm, g_hbm,
          idx, bufs, bufd, sem0, sem1, sem2, sem3, wsem):
        cid = lax.axis_index("c")
        sid = lax.axis_index("s")
        wid = sid * NC + cid
        base = wid * chunk
        gsems = [sem0, sem1, sem2, sem3]
        ic0 = pltpu.async_copy(idx_hbm.at[pl.ds(nj * wid, nj)],
                               idx.at[pl.ds(0, nj)], wsem)
        ic1 = pltpu.async_copy(idx_hbm.at[pl.ds(nr + nj * wid, nj)],
                               idx.at[pl.ds(nj, nj)], wsem)
        ic0.wait()
        ic1.wait()
        gets = [pltpu.async_copy(tab_hbm.at[idx.at[j]],
                                 (bufs if j < nj else bufd)
                                 .at[pl.ds((j % nj) * 128, 128)],
                                 gsems[j])
                for j in range(2 * nj)]
        puts = []
        for j in range(nj):
            gets[j].wait()
            gets[nj + j].wait()

            def body(r, _, j=j):
                row = j * 128 + r
                for l in range(M // 16):
                    sl = pl.ds(l * 16, 16)
                    bufs[row, sl] = bufs[row, sl] + bufd[row, sl]
                return 0

            lax.fori_loop(0, 128, body, 0)
            puts.append(pltpu.async_copy(
                bufs.at[pl.ds(j * 128, 128)],
                g_hbm.at[pl.ds(base + j * 128, 128)], wsem))
        for p in puts:
            p.wait()

    return k(tab, idxcat)


def kernel(coords, hidden, edges, Wm1, bm1, Wm2, bm2, Wc1, bc1, Wc2,
           Wh1, bh1, Wh2, bh2):
    e = edges.shape[1]
    ef = B * e

    srcf = edges[0].astype(jnp.int32)[:, None]
    dstf = edges[1].astype(jnp.int32)[:, None]
    srcrow = edges[0].astype(jnp.int32)[None, :]
    dstrow = edges[1].astype(jnp.int32)[None, :]
    er = e // 128
    srcr = edges[0].astype(jnp.int32).reshape(er, 128)
    dstr = edges[1].astype(jnp.int32).reshape(er, 128)
    boff = ((jnp.arange(B * er, dtype=jnp.int32) // er) * N)[:, None]
    idxcat = jnp.concatenate([jnp.tile(srcr, (B, 1)) + boff,
                              jnp.tile(dstr, (B, 1)) + boff + NN], axis=0)

    tab = pl.pallas_call(
        _tab_body,
        out_shape=jax.ShapeDtypeStruct((2 * NN, M), F32),
    )(hidden, Wm1)

    g = _sc_gather(tab, idxcat, ef)

    co, ho = pl.pallas_call(
        _edge_body,
        out_shape=(jax.ShapeDtypeStruct((B, N, 3), F32),
                   jax.ShapeDtypeStruct((B, N, H), F32)),
    )(g, srcf, dstf, srcrow, dstrow, coords, hidden,
      Wm1[0:1], bm1[None, :], Wm2, bm2[None, :], Wc1, bc1[None, :],
      Wc2.reshape(1, M), Wh1[:H], Wh1[H:], bh1[None, :], Wh2, bh2[None, :])

    return co, ho
